# baseline (speedup 1.0000x reference)
import functools

import jax
import jax.numpy as jnp
from jax import lax
from jax.experimental import pallas as pl
from jax.experimental.pallas import tpu as pltpu
from jax.experimental.pallas import tpu_sc as plsc

NUM_ROWS = 1000000
DIM = 32

NC = 2
NS = 16
NW = NC * NS

BATCH = 16384
SEQ = 50
BW = BATCH // NW
JV = BW // 16


def _body(idx_hbm, table_hbm, out_hbm, idx_v, idxs_v, rows_v, trans_v,
          gsem, psem):
    wid = lax.axis_index("s") * NC + lax.axis_index("c")
    b0 = wid * BW

    pltpu.sync_copy(idx_hbm.at[pl.ds(b0, BW), :], idx_v)

    iota = lax.iota(jnp.int32, 16)

    def repack(s, ib):
        col = jnp.full((16,), 0, jnp.int32) + s

        @plsc.parallel_loop(0, JV, unroll=4)
        def _jv(jv):
            row = iota + (jv * 16)
            vec = plsc.load_gather(idx_v, [row, col])
            idxs_v[ib, pl.ds(jv * 16, 16)] = vec

    def gather(ib, rb):
        src = table_hbm.at[idxs_v.at[ib]]
        return pltpu.make_async_copy(src, rows_v.at[rb], gsem.at[rb])

    def writeback(s, tb):
        dst = out_hbm.at[s, :, pl.ds(b0, BW)]
        return pltpu.make_async_copy(trans_v.at[tb], dst, psem.at[tb])

    def transpose(rb, tb):
        rows = rows_v.at[rb]
        trans = trans_v.at[tb]

        @plsc.parallel_loop(0, DIM, unroll=4)
        def _d(d):
            col = jnp.full((16,), 0, jnp.int32) + d
            for jv in range(JV):
                row = iota + (jv * 16)
                vec = plsc.load_gather(rows, [row, col])
                trans[d, pl.ds(jv * 16, 16)] = vec

    repack(0, 0)
    gather(0, 0).start()

    @pl.loop(0, SEQ, step=2)
    def _s2(s0):
        for h in range(2):
            s = s0 + h
            rb = h
            tb = h
            ib = h
            gather(ib, rb).wait()

            @pl.when(s + 1 < SEQ)
            def _():
                repack(s + 1, 1 - ib)
                gather(1 - ib, 1 - rb).start()

            @pl.when(s >= 2)
            def _():
                writeback(s - 2, tb).wait()

            transpose(rb, tb)
            writeback(s, tb).start()

    for tb in range(2):
        writeback(0, tb).wait()


@jax.jit
def _lookup(token_ids, weight):
    mesh = plsc.VectorSubcoreMesh(core_axis_name="c", subcore_axis_name="s")
    f = functools.partial(
        pl.kernel,
        out_type=jax.ShapeDtypeStruct((SEQ, DIM, BATCH), jnp.float32),
        mesh=mesh,
        scratch_types=[
            pltpu.VMEM((BW, SEQ), jnp.int32),
            pltpu.VMEM((2, BW), jnp.int32),
            pltpu.VMEM((2, BW, DIM), jnp.float32),
            pltpu.VMEM((2, DIM, BW), jnp.float32),
            pltpu.SemaphoreType.DMA((2,)),
            pltpu.SemaphoreType.DMA((2,)),
        ],
        compiler_params=pltpu.CompilerParams(
            use_tc_tiling_on_sc=False, needs_layout_passes=False
        ),
    )(_body)
    return f(token_ids, weight)


def kernel(token_ids, weight):
    out = _lookup(token_ids.astype(jnp.int32), weight)
    return out.transpose(2, 0, 1)


# speedup vs baseline: 1.0264x; 1.0264x over previous
import functools

import jax
import jax.numpy as jnp
from jax import lax
from jax.experimental import pallas as pl
from jax.experimental.pallas import tpu as pltpu
from jax.experimental.pallas import tpu_sc as plsc

NUM_ROWS = 1000000
DIM = 32

NC = 2
NS = 16
NW = NC * NS

BATCH = 16384
SEQ = 50
BW = BATCH // NW
JV = BW // 16
CT = BW // 128


def _body(idx_hbm, table_hbm, out_hbm, idx_v, idxs_v, rows_v, trans_v,
          gsem, psem):
    wid = lax.axis_index("s") * NC + lax.axis_index("c")
    b0 = wid * BW

    pltpu.sync_copy(idx_hbm.at[pl.ds(b0, BW), :], idx_v)

    iota = lax.iota(jnp.int32, 16)

    def repack(s, ib):
        col = jnp.full((16,), 0, jnp.int32) + s

        @plsc.parallel_loop(0, JV, unroll=4)
        def _jv(jv):
            row = iota + (jv * 16)
            vec = plsc.load_gather(idx_v, [row, col])
            idxs_v[ib, pl.ds(jv * 16, 16)] = vec

    def gather(ib, rb):
        src = table_hbm.at[idxs_v.at[ib]]
        return pltpu.make_async_copy(src, rows_v.at[rb], gsem.at[rb])

    def writeback(s, tb):
        dst = out_hbm.at[s, :, pl.ds(CT * wid, CT), :, :]
        return pltpu.make_async_copy(trans_v.at[tb], dst, psem.at[tb])

    def transpose(rb, tb):
        rows = rows_v.at[rb]
        trans = trans_v.at[tb]

        @plsc.parallel_loop(0, JV, unroll=4)
        def _jv(jv):
            row = iota + (jv * 16)
            ctl = jv // 8
            cv = jv % 8
            for dt in range(DIM // 8):
                for dp in range(8):
                    col = jnp.full((16,), 0, jnp.int32) + (dt * 8 + dp)
                    vec = plsc.load_gather(rows, [row, col])
                    trans[dt, ctl, dp, pl.ds(cv * 16, 16)] = vec

    repack(0, 0)
    gather(0, 0).start()

    @pl.loop(0, SEQ, step=2)
    def _s2(s0):
        for h in range(2):
            s = s0 + h
            rb = h
            tb = h
            ib = h
            gather(ib, rb).wait()

            @pl.when(s + 1 < SEQ)
            def _():
                repack(s + 1, 1 - ib)
                gather(1 - ib, 1 - rb).start()

            @pl.when(s >= 2)
            def _():
                writeback(s - 2, tb).wait()

            transpose(rb, tb)
            writeback(s, tb).start()

    for tb in range(2):
        writeback(0, tb).wait()


@jax.jit
def _lookup(token_ids, weight):
    mesh = plsc.VectorSubcoreMesh(core_axis_name="c", subcore_axis_name="s")
    f = functools.partial(
        pl.kernel,
        out_type=jax.ShapeDtypeStruct(
            (SEQ, DIM // 8, BATCH // 128, 8, 128), jnp.float32
        ),
        mesh=mesh,
        scratch_types=[
            pltpu.VMEM((BW, SEQ), jnp.int32),
            pltpu.VMEM((2, BW), jnp.int32),
            pltpu.VMEM((2, BW, DIM), jnp.float32),
            pltpu.VMEM((2, DIM // 8, CT, 8, 128), jnp.float32),
            pltpu.SemaphoreType.DMA((2,)),
            pltpu.SemaphoreType.DMA((2,)),
        ],
        compiler_params=pltpu.CompilerParams(
            use_tc_tiling_on_sc=False, needs_layout_passes=False
        ),
    )(_body)
    return f(token_ids, weight)


def kernel(token_ids, weight):
    out5 = _lookup(token_ids.astype(jnp.int32), weight)
    return out5.transpose(2, 4, 0, 1, 3).reshape(BATCH, SEQ, DIM)


# speedup vs baseline: 1.1296x; 1.1006x over previous
import functools

import jax
import jax.numpy as jnp
from jax import lax
from jax.experimental import pallas as pl
from jax.experimental.pallas import tpu as pltpu
from jax.experimental.pallas import tpu_sc as plsc

NUM_ROWS = 1000000
DIM = 32

NC = 2
NS = 16
NW = NC * NS

BATCH = 16384
SEQ = 50
BW = BATCH // NW
JV = BW // 16
CT = BW // 128


def _body(idx_hbm, table_hbm, out_hbm, idx_v, idxs_v, rows_v, trans_v,
          gsem, psem):
    wid = lax.axis_index("s") * NC + lax.axis_index("c")
    b0 = wid * BW

    pltpu.sync_copy(idx_hbm.at[pl.ds(b0, BW), :], idx_v)

    iota = lax.iota(jnp.int32, 16)

    def repack(s, ib):
        col = jnp.full((16,), 0, jnp.int32) + s

        @plsc.parallel_loop(0, JV, unroll=4)
        def _jv(jv):
            row = iota + (jv * 16)
            vec = plsc.load_gather(idx_v, [row, col])
            idxs_v[ib, pl.ds(jv * 16, 16)] = vec

    def gather(ib, rb):
        src = table_hbm.at[idxs_v.at[ib]]
        return pltpu.make_async_copy(src, rows_v.at[rb], gsem.at[rb])

    def writeback(s, tb):
        dst = out_hbm.at[s, :, pl.ds(CT * wid, CT), :, :]
        return pltpu.make_async_copy(trans_v.at[tb], dst, psem.at[tb])

    def transpose(rb, tb):
        rows = rows_v.at[rb]
        trans = trans_v.at[tb]

        @plsc.parallel_loop(0, DIM, unroll=4)
        def _d(d):
            dt = d // 8
            dp = d - dt * 8
            col = jnp.full((16,), 0, jnp.int32) + d
            for jv in range(JV):
                row = iota + (jv * 16)
                vec = plsc.load_gather(rows, [row, col])
                trans[dt, jv // 8, dp, pl.ds((jv % 8) * 16, 16)] = vec

    repack(0, 0)
    gather(0, 0).start()

    @pl.loop(0, SEQ, step=2)
    def _s2(s0):
        for h in range(2):
            s = s0 + h
            rb = h
            tb = h
            ib = h
            gather(ib, rb).wait()

            @pl.when(s + 1 < SEQ)
            def _():
                repack(s + 1, 1 - ib)
                gather(1 - ib, 1 - rb).start()

            @pl.when(s >= 2)
            def _():
                writeback(s - 2, tb).wait()

            transpose(rb, tb)
            writeback(s, tb).start()

    for tb in range(2):
        writeback(0, tb).wait()


@jax.jit
def _lookup(token_ids, weight):
    mesh = plsc.VectorSubcoreMesh(core_axis_name="c", subcore_axis_name="s")
    f = functools.partial(
        pl.kernel,
        out_type=jax.ShapeDtypeStruct(
            (SEQ, DIM // 8, BATCH // 128, 8, 128), jnp.float32
        ),
        mesh=mesh,
        scratch_types=[
            pltpu.VMEM((BW, SEQ), jnp.int32),
            pltpu.VMEM((2, BW), jnp.int32),
            pltpu.VMEM((2, BW, DIM), jnp.float32),
            pltpu.VMEM((2, DIM // 8, CT, 8, 128), jnp.float32),
            pltpu.SemaphoreType.DMA((2,)),
            pltpu.SemaphoreType.DMA((2,)),
        ],
        compiler_params=pltpu.CompilerParams(
            use_tc_tiling_on_sc=False, needs_layout_passes=False
        ),
    )(_body)
    return f(token_ids, weight)


def kernel(token_ids, weight):
    out5 = _lookup(token_ids.astype(jnp.int32), weight)
    return out5.transpose(2, 4, 0, 1, 3).reshape(BATCH, SEQ, DIM)
